# Initial kernel scaffold; baseline (speedup 1.0000x reference)
import functools

import jax
import jax.numpy as jnp
from jax import lax
from jax.experimental import pallas as pl
from jax.experimental.pallas import tpu as pltpu
from jax.experimental.pallas import tpu_sc as plsc

FDIM = 128
LANES = 16
NC = 2
NS = 16
NW = NC * NS
EB = 128
NPAD = 10240
ROWS_PER_W = NPAD // NS
BM = 512


def _sc_degree(dst3d):
    C = dst3d.shape[1]
    mesh = plsc.VectorSubcoreMesh(core_axis_name="c", subcore_axis_name="s")

    @functools.partial(
        pl.kernel,
        mesh=mesh,
        out_type=(
            jax.ShapeDtypeStruct((NPAD, LANES), jnp.float32),
            jax.ShapeDtypeStruct((NPAD, LANES), jnp.float32),
        ),
        scratch_types=(
            pltpu.VMEM((C, EB), jnp.int32),
            pltpu.VMEM((EB, LANES), jnp.float32),
            pltpu.VMEM((64, LANES), jnp.float32),
            pltpu.VMEM_SHARED((NPAD, LANES), jnp.float32),
        ),
    )
    def k(dst_hbm, deg0_hbm, deg1_hbm, dst_v, ones_v, zb_v, deg_sh):
        c = lax.axis_index("c")
        s = lax.axis_index("s")
        wid = s * NC + c
        pltpu.sync_copy(dst_hbm.at[wid], dst_v)
        one = jnp.full((LANES,), 1.0, jnp.float32)
        zero = jnp.zeros((LANES,), jnp.float32)
        for r in range(EB):
            ones_v[r] = one
        for r in range(64):
            zb_v[r] = zero
        base = s * ROWS_PER_W
        for t in range(ROWS_PER_W // 64):
            pltpu.sync_copy(zb_v, deg_sh.at[pl.ds(base + t * 64, 64)])
        plsc.subcore_barrier()

        def body(j, carry):
            pltpu.sync_copy(ones_v, deg_sh.at[dst_v.at[j]], add=True)
            return carry

        lax.fori_loop(0, C, body, 0)
        plsc.subcore_barrier()
        rows = pl.ds(base, ROWS_PER_W)

        @pl.when(c == 0)
        def _():
            pltpu.sync_copy(deg_sh.at[rows], deg0_hbm.at[rows])

        @pl.when(c == 1)
        def _():
            pltpu.sync_copy(deg_sh.at[rows], deg1_hbm.at[rows])

    return k(dst3d)


def _sc_accumulate(src3d, dst3d, g_pad, zrows):
    C = src3d.shape[1]
    mesh = plsc.VectorSubcoreMesh(core_axis_name="c", subcore_axis_name="s")

    @functools.partial(
        pl.kernel,
        mesh=mesh,
        out_type=(
            jax.ShapeDtypeStruct((NPAD, FDIM), jnp.float32),
            jax.ShapeDtypeStruct((NPAD, FDIM), jnp.float32),
        ),
        scratch_types=(
            pltpu.VMEM((C, EB), jnp.int32),
            pltpu.VMEM((C, EB), jnp.int32),
            pltpu.VMEM((2, EB, FDIM), jnp.float32),
            pltpu.VMEM_SHARED((NPAD, FDIM), jnp.float32),
            pltpu.SemaphoreType.DMA,
            pltpu.SemaphoreType.DMA,
        ),
    )
    def k(src_hbm, dst_hbm, g_hbm, z_hbm, acc0_hbm, acc1_hbm,
          src_v, dst_v, gbuf, acc_sh, sem0, sem1):
        c = lax.axis_index("c")
        s = lax.axis_index("s")
        wid = s * NC + c
        pltpu.sync_copy(src_hbm.at[wid], src_v)
        pltpu.sync_copy(dst_hbm.at[wid], dst_v)
        base = s * ROWS_PER_W
        rows = pl.ds(base, ROWS_PER_W)
        pltpu.sync_copy(z_hbm.at[rows], acc_sh.at[rows])
        plsc.subcore_barrier()

        sems = (sem0, sem1)
        for b in (0, 1):
            pltpu.async_copy(g_hbm.at[src_v.at[b]], gbuf.at[b], sems[b])

        def body(i, carry):
            for b in (0, 1):
                j = i * 2 + b
                pltpu.make_async_copy(
                    g_hbm.at[src_v.at[j]], gbuf.at[b], sems[b]).wait()
                pltpu.sync_copy(gbuf.at[b], acc_sh.at[dst_v.at[j]], add=True)
                nxt = j + 2

                @pl.when(nxt < C)
                def _():
                    pltpu.async_copy(
                        g_hbm.at[src_v.at[nxt]], gbuf.at[b], sems[b])
            return carry

        lax.fori_loop(0, C // 2, body, 0)
        plsc.subcore_barrier()

        @pl.when(c == 0)
        def _():
            pltpu.sync_copy(acc_sh.at[rows], acc0_hbm.at[rows])

        @pl.when(c == 1)
        def _():
            pltpu.sync_copy(acc_sh.at[rows], acc1_hbm.at[rows])

    return k(src3d, dst3d, g_pad, zrows)


def _dinv(d0_ref, d1_ref):
    d = d0_ref[:, 0:1] + d1_ref[:, 0:1] + 1.0
    return lax.rsqrt(d)


def _tc_first(xp, W1, deg0, deg1):
    def body(x_ref, w_ref, d0_ref, d1_ref, g_ref):
        h = jnp.dot(x_ref[...], w_ref[...],
                    preferred_element_type=jnp.float32)
        g_ref[...] = h * _dinv(d0_ref, d1_ref)

    return pl.pallas_call(
        body,
        grid=(NPAD // BM,),
        in_specs=[
            pl.BlockSpec((BM, FDIM), lambda i: (i, 0)),
            pl.BlockSpec((FDIM, FDIM), lambda i: (0, 0)),
            pl.BlockSpec((BM, LANES), lambda i: (i, 0)),
            pl.BlockSpec((BM, LANES), lambda i: (i, 0)),
        ],
        out_specs=pl.BlockSpec((BM, FDIM), lambda i: (i, 0)),
        out_shape=jax.ShapeDtypeStruct((NPAD, FDIM), jnp.float32),
    )(xp, W1, deg0, deg1)


def _tc_mid(a0, a1, g1, deg0, deg1, b1, W2):
    def body(a0_ref, a1_ref, g_ref, d0_ref, d1_ref, b_ref, w_ref, o_ref):
        dinv = _dinv(d0_ref, d1_ref)
        z = dinv * (a0_ref[...] + a1_ref[...] + g_ref[...]) + b_ref[...]
        z = jnp.maximum(z, 0.0)
        h = jnp.dot(z, w_ref[...], preferred_element_type=jnp.float32)
        o_ref[...] = h * dinv

    return pl.pallas_call(
        body,
        grid=(NPAD // BM,),
        in_specs=[
            pl.BlockSpec((BM, FDIM), lambda i: (i, 0)),
            pl.BlockSpec((BM, FDIM), lambda i: (i, 0)),
            pl.BlockSpec((BM, FDIM), lambda i: (i, 0)),
            pl.BlockSpec((BM, LANES), lambda i: (i, 0)),
            pl.BlockSpec((BM, LANES), lambda i: (i, 0)),
            pl.BlockSpec((1, FDIM), lambda i: (0, 0)),
            pl.BlockSpec((FDIM, FDIM), lambda i: (0, 0)),
        ],
        out_specs=pl.BlockSpec((BM, FDIM), lambda i: (i, 0)),
        out_shape=jax.ShapeDtypeStruct((NPAD, FDIM), jnp.float32),
    )(a0, a1, g1, deg0, deg1, b1.reshape(1, FDIM), W2)


def _tc_last(a0, a1, g2, deg0, deg1, b2, Wfc, bfc):
    def body(a0_ref, a1_ref, g_ref, d0_ref, d1_ref, b_ref, w_ref, bf_ref,
             o_ref):
        dinv = _dinv(d0_ref, d1_ref)
        z = dinv * (a0_ref[...] + a1_ref[...] + g_ref[...]) + b_ref[...]
        z = jnp.maximum(z, 0.0)
        o_ref[...] = jnp.sum(z * w_ref[...], axis=1) + bf_ref[0, 0]

    return pl.pallas_call(
        body,
        grid=(NPAD // BM,),
        in_specs=[
            pl.BlockSpec((BM, FDIM), lambda i: (i, 0)),
            pl.BlockSpec((BM, FDIM), lambda i: (i, 0)),
            pl.BlockSpec((BM, FDIM), lambda i: (i, 0)),
            pl.BlockSpec((BM, LANES), lambda i: (i, 0)),
            pl.BlockSpec((BM, LANES), lambda i: (i, 0)),
            pl.BlockSpec((1, FDIM), lambda i: (0, 0)),
            pl.BlockSpec((1, FDIM), lambda i: (0, 0)),
            pl.BlockSpec((1, 1), lambda i: (0, 0)),
        ],
        out_specs=pl.BlockSpec((BM,), lambda i: (i,)),
        out_shape=jax.ShapeDtypeStruct((NPAD,), jnp.float32),
    )(a0, a1, g2, deg0, deg1, b2.reshape(1, FDIM), Wfc.reshape(1, FDIM),
      bfc.reshape(1, 1))


def kernel(x, edge_index, W1, b1, W2, b2, Wfc, bfc):
    n = x.shape[0]
    e = edge_index.shape[1]
    C = -(-e // (NW * EB))
    C += C % 2
    epad = NW * C * EB

    pad_idx = jnp.full((epad - e,), NPAD - 1, jnp.int32)
    src3d = jnp.concatenate([edge_index[0], pad_idx]).reshape(NW, C, EB)
    dst3d = jnp.concatenate([edge_index[1], pad_idx]).reshape(NW, C, EB)
    xp = jnp.pad(x, ((0, NPAD - n), (0, 0)))
    zrows = jnp.zeros((NPAD, FDIM), jnp.float32)

    deg0, deg1 = _sc_degree(dst3d)
    g1 = _tc_first(xp, W1, deg0, deg1)
    a0, a1 = _sc_accumulate(src3d, dst3d, g1, zrows)
    g2 = _tc_mid(a0, a1, g1, deg0, deg1, b1, W2)
    a0, a1 = _sc_accumulate(src3d, dst3d, g2, zrows)
    outp = _tc_last(a0, a1, g2, deg0, deg1, b2, Wfc, bfc)
    return outp[:n]


# speedup vs baseline: 7.9837x; 7.9837x over previous
import functools

import jax
import jax.numpy as jnp
from jax import lax
from jax.experimental import pallas as pl
from jax.experimental.pallas import tpu as pltpu
from jax.experimental.pallas import tpu_sc as plsc

FDIM = 128
LANES = 16
NC = 2
NS = 16
NW = NC * NS
EB = 128
NPAD = 10240
ROWS_PER_W = NPAD // NS
BM = 512


def _sc_degree(dst3d):
    C = dst3d.shape[1]
    mesh = plsc.VectorSubcoreMesh(core_axis_name="c", subcore_axis_name="s")

    @functools.partial(
        pl.kernel,
        mesh=mesh,
        out_type=(
            jax.ShapeDtypeStruct((NPAD, LANES), jnp.float32),
            jax.ShapeDtypeStruct((NPAD, LANES), jnp.float32),
        ),
        scratch_types=(
            pltpu.VMEM((C, EB), jnp.int32),
            pltpu.VMEM((EB, LANES), jnp.float32),
            pltpu.VMEM((64, LANES), jnp.float32),
            pltpu.VMEM_SHARED((NPAD, LANES), jnp.float32),
        ),
    )
    def k(dst_hbm, deg0_hbm, deg1_hbm, dst_v, ones_v, zb_v, deg_sh):
        c = lax.axis_index("c")
        s = lax.axis_index("s")
        wid = s * NC + c
        pltpu.sync_copy(dst_hbm.at[wid], dst_v)
        one = jnp.full((LANES,), 1.0, jnp.float32)
        zero = jnp.zeros((LANES,), jnp.float32)
        for r in range(EB):
            ones_v[r] = one
        for r in range(64):
            zb_v[r] = zero
        base = s * ROWS_PER_W
        for t in range(ROWS_PER_W // 64):
            pltpu.sync_copy(zb_v, deg_sh.at[pl.ds(base + t * 64, 64)])
        plsc.subcore_barrier()

        def body(j, carry):
            pltpu.sync_copy(ones_v, deg_sh.at[dst_v.at[j]], add=True)
            return carry

        lax.fori_loop(0, C, body, 0)
        plsc.subcore_barrier()
        rows = pl.ds(base, ROWS_PER_W)

        @pl.when(c == 0)
        def _():
            pltpu.sync_copy(deg_sh.at[rows], deg0_hbm.at[rows])

        @pl.when(c == 1)
        def _():
            pltpu.sync_copy(deg_sh.at[rows], deg1_hbm.at[rows])

    return k(dst3d)


def _sc_accumulate(src3d, dst3d, g_pad, zrows):
    C = src3d.shape[1]
    mesh = plsc.VectorSubcoreMesh(core_axis_name="c", subcore_axis_name="s")

    CP = C // 2

    @functools.partial(
        pl.kernel,
        mesh=mesh,
        out_type=(
            jax.ShapeDtypeStruct((NPAD, FDIM), jnp.float32),
            jax.ShapeDtypeStruct((NPAD, FDIM), jnp.float32),
        ),
        scratch_types=(
            pltpu.VMEM((CP, EB), jnp.int32),
            pltpu.VMEM((CP, EB), jnp.int32),
            pltpu.VMEM((2, EB, FDIM), jnp.float32),
            pltpu.VMEM_SHARED((NPAD, FDIM), jnp.float32),
            pltpu.SemaphoreType.DMA,
            pltpu.SemaphoreType.DMA,
        ),
    )
    def k(src_hbm, dst_hbm, g_hbm, z_hbm, acc0_hbm, acc1_hbm,
          src_v, dst_v, gbuf, acc_sh, sem0, sem1):
        c = lax.axis_index("c")
        s = lax.axis_index("s")
        wid = s * NC + c
        base = s * ROWS_PER_W
        rows = pl.ds(base, ROWS_PER_W)
        pltpu.sync_copy(z_hbm.at[rows], acc_sh.at[rows])
        plsc.subcore_barrier()

        sems = (sem0, sem1)
        for p in range(2):
            pltpu.sync_copy(src_hbm.at[wid, pl.ds(p * CP, CP)], src_v)
            pltpu.sync_copy(dst_hbm.at[wid, pl.ds(p * CP, CP)], dst_v)
            for b in (0, 1):
                pltpu.async_copy(g_hbm.at[src_v.at[b]], gbuf.at[b], sems[b])

            def body(i, carry):
                for b in (0, 1):
                    j = i * 2 + b
                    pltpu.make_async_copy(
                        g_hbm.at[src_v.at[j]], gbuf.at[b], sems[b]).wait()
                    pltpu.sync_copy(
                        gbuf.at[b], acc_sh.at[dst_v.at[j]], add=True)
                    nxt = j + 2

                    @pl.when(nxt < CP)
                    def _():
                        pltpu.async_copy(
                            g_hbm.at[src_v.at[nxt]], gbuf.at[b], sems[b])
                return carry

            lax.fori_loop(0, CP // 2, body, 0)
        plsc.subcore_barrier()

        @pl.when(c == 0)
        def _():
            pltpu.sync_copy(acc_sh.at[rows], acc0_hbm.at[rows])

        @pl.when(c == 1)
        def _():
            pltpu.sync_copy(acc_sh.at[rows], acc1_hbm.at[rows])

    return k(src3d, dst3d, g_pad, zrows)


def _dinv(d0_ref, d1_ref):
    d = d0_ref[:, 0:1] + d1_ref[:, 0:1] + 1.0
    return lax.rsqrt(d)


def _tc_first(xp, W1, deg0, deg1):
    def body(x_ref, w_ref, d0_ref, d1_ref, g_ref):
        h = jnp.dot(x_ref[...], w_ref[...],
                    preferred_element_type=jnp.float32)
        g_ref[...] = h * _dinv(d0_ref, d1_ref)

    return pl.pallas_call(
        body,
        grid=(NPAD // BM,),
        in_specs=[
            pl.BlockSpec((BM, FDIM), lambda i: (i, 0)),
            pl.BlockSpec((FDIM, FDIM), lambda i: (0, 0)),
            pl.BlockSpec((BM, LANES), lambda i: (i, 0)),
            pl.BlockSpec((BM, LANES), lambda i: (i, 0)),
        ],
        out_specs=pl.BlockSpec((BM, FDIM), lambda i: (i, 0)),
        out_shape=jax.ShapeDtypeStruct((NPAD, FDIM), jnp.float32),
    )(xp, W1, deg0, deg1)


def _tc_mid(a0, a1, g1, deg0, deg1, b1, W2):
    def body(a0_ref, a1_ref, g_ref, d0_ref, d1_ref, b_ref, w_ref, o_ref):
        dinv = _dinv(d0_ref, d1_ref)
        z = dinv * (a0_ref[...] + a1_ref[...] + g_ref[...]) + b_ref[...]
        z = jnp.maximum(z, 0.0)
        h = jnp.dot(z, w_ref[...], preferred_element_type=jnp.float32)
        o_ref[...] = h * dinv

    return pl.pallas_call(
        body,
        grid=(NPAD // BM,),
        in_specs=[
            pl.BlockSpec((BM, FDIM), lambda i: (i, 0)),
            pl.BlockSpec((BM, FDIM), lambda i: (i, 0)),
            pl.BlockSpec((BM, FDIM), lambda i: (i, 0)),
            pl.BlockSpec((BM, LANES), lambda i: (i, 0)),
            pl.BlockSpec((BM, LANES), lambda i: (i, 0)),
            pl.BlockSpec((1, FDIM), lambda i: (0, 0)),
            pl.BlockSpec((FDIM, FDIM), lambda i: (0, 0)),
        ],
        out_specs=pl.BlockSpec((BM, FDIM), lambda i: (i, 0)),
        out_shape=jax.ShapeDtypeStruct((NPAD, FDIM), jnp.float32),
    )(a0, a1, g1, deg0, deg1, b1.reshape(1, FDIM), W2)


def _tc_last(a0, a1, g2, deg0, deg1, b2, Wfc, bfc):
    def body(a0_ref, a1_ref, g_ref, d0_ref, d1_ref, b_ref, w_ref, bf_ref,
             o_ref):
        dinv = _dinv(d0_ref, d1_ref)
        z = dinv * (a0_ref[...] + a1_ref[...] + g_ref[...]) + b_ref[...]
        z = jnp.maximum(z, 0.0)
        o_ref[...] = jnp.sum(z * w_ref[...], axis=1) + bf_ref[0, 0]

    return pl.pallas_call(
        body,
        grid=(NPAD // BM,),
        in_specs=[
            pl.BlockSpec((BM, FDIM), lambda i: (i, 0)),
            pl.BlockSpec((BM, FDIM), lambda i: (i, 0)),
            pl.BlockSpec((BM, FDIM), lambda i: (i, 0)),
            pl.BlockSpec((BM, LANES), lambda i: (i, 0)),
            pl.BlockSpec((BM, LANES), lambda i: (i, 0)),
            pl.BlockSpec((1, FDIM), lambda i: (0, 0)),
            pl.BlockSpec((1, FDIM), lambda i: (0, 0)),
            pl.BlockSpec((1, 1), lambda i: (0, 0)),
        ],
        out_specs=pl.BlockSpec((BM,), lambda i: (i,)),
        out_shape=jax.ShapeDtypeStruct((NPAD,), jnp.float32),
    )(a0, a1, g2, deg0, deg1, b2.reshape(1, FDIM), Wfc.reshape(1, FDIM),
      bfc.reshape(1, 1))


def kernel(x, edge_index, W1, b1, W2, b2, Wfc, bfc):
    n = x.shape[0]
    e = edge_index.shape[1]
    C = -(-e // (NW * EB))
    C = (C + 3) // 4 * 4
    epad = NW * C * EB

    pad_idx = jnp.full((epad - e,), NPAD - 1, jnp.int32)
    src3d = jnp.concatenate([edge_index[0], pad_idx]).reshape(NW, C, EB)
    dst3d = jnp.concatenate([edge_index[1], pad_idx]).reshape(NW, C, EB)
    xp = jnp.pad(x, ((0, NPAD - n), (0, 0)))
    zrows = jnp.zeros((NPAD, FDIM), jnp.float32)

    deg0, deg1 = _sc_degree(dst3d)
    g1 = _tc_first(xp, W1, deg0, deg1)
    a0, a1 = _sc_accumulate(src3d, dst3d, g1, zrows)
    g2 = _tc_mid(a0, a1, g1, deg0, deg1, b1, W2)
    a0, a1 = _sc_accumulate(src3d, dst3d, g2, zrows)
    outp = _tc_last(a0, a1, g2, deg0, deg1, b2, Wfc, bfc)
    return outp[:n]


# speedup vs baseline: 9.3666x; 1.1732x over previous
import functools

import jax
import jax.numpy as jnp
from jax import lax
from jax.experimental import pallas as pl
from jax.experimental.pallas import tpu as pltpu
from jax.experimental.pallas import tpu_sc as plsc

FDIM = 128
LANES = 16
NC = 2
NS = 16
NW = NC * NS
EB = 64
NB = 4
CP = 32
SC1_FRAC = 0.2
NPAD = 10240
ROWS_PER_W = NPAD // NS
BM = 512


def _sc_degree(dst3d, ones_blk, zrows):
    DPH = dst3d.shape[1] // (2 * CP)
    mesh = plsc.VectorSubcoreMesh(core_axis_name="c", subcore_axis_name="s")

    @functools.partial(
        pl.kernel,
        mesh=mesh,
        out_type=(
            jax.ShapeDtypeStruct((NPAD, FDIM), jnp.float32),
            jax.ShapeDtypeStruct((NPAD, FDIM), jnp.float32),
        ),
        scratch_types=(
            pltpu.VMEM((CP, EB), jnp.int32),
            pltpu.VMEM((EB, FDIM), jnp.float32),
            pltpu.VMEM_SHARED((NPAD, FDIM), jnp.float32),
        ),
    )
    def k(dst_hbm, ones_hbm, z_hbm, deg0_hbm, deg1_hbm, dst_v, obuf,
          deg_sh):
        c = lax.axis_index("c")
        s = lax.axis_index("s")
        base = s * ROWS_PER_W
        rows = pl.ds(base, ROWS_PER_W)
        pltpu.sync_copy(z_hbm.at[rows], deg_sh.at[rows])
        pltpu.sync_copy(ones_hbm, obuf)
        plsc.subcore_barrier()

        for p in range(DPH):
            pltpu.sync_copy(
                dst_hbm.at[s, pl.ds((c * DPH + p) * CP, CP)], dst_v)

            def body(j, carry):
                pltpu.sync_copy(obuf, deg_sh.at[dst_v.at[j]], add=True)
                return carry

            lax.fori_loop(0, CP, body, 0)
        plsc.subcore_barrier()

        @pl.when(c == 0)
        def _():
            pltpu.sync_copy(deg_sh.at[rows], deg0_hbm.at[rows])

        @pl.when(c == 1)
        def _():
            pltpu.sync_copy(deg_sh.at[rows], deg1_hbm.at[rows])

    return k(dst3d, ones_blk, zrows)


def _sc_accumulate(src0, dst0, src1, dst1, g_pad, zrows):
    C0 = src0.shape[1]
    C1 = src1.shape[1]
    mesh = plsc.VectorSubcoreMesh(core_axis_name="c", subcore_axis_name="s")

    @functools.partial(
        pl.kernel,
        mesh=mesh,
        out_type=(
            jax.ShapeDtypeStruct((NPAD, FDIM), jnp.float32),
            jax.ShapeDtypeStruct((NPAD, FDIM), jnp.float32),
        ),
        scratch_types=(
            pltpu.VMEM((CP, EB), jnp.int32),
            pltpu.VMEM((CP, EB), jnp.int32),
            pltpu.VMEM((NB, EB, FDIM), jnp.float32),
            pltpu.VMEM_SHARED((NPAD, FDIM), jnp.float32),
        ) + (pltpu.SemaphoreType.DMA,) * NB,
    )
    def k(src0_hbm, dst0_hbm, src1_hbm, dst1_hbm, g_hbm, z_hbm,
          acc0_hbm, acc1_hbm, src_v, dst_v, gbuf, acc_sh, *sems):
        c = lax.axis_index("c")
        s = lax.axis_index("s")
        base = s * ROWS_PER_W
        rows = pl.ds(base, ROWS_PER_W)
        pltpu.sync_copy(z_hbm.at[rows], acc_sh.at[rows])
        plsc.subcore_barrier()

        def run_loops(src_hbm, dst_hbm, nph):
            for p in range(nph):
                pltpu.sync_copy(src_hbm.at[s, pl.ds(p * CP, CP)], src_v)
                pltpu.sync_copy(dst_hbm.at[s, pl.ds(p * CP, CP)], dst_v)
                for b in range(NB):
                    pltpu.async_copy(
                        g_hbm.at[src_v.at[b]], gbuf.at[b], sems[b])

                def body(i, carry):
                    for b in range(NB):
                        j = i * NB + b
                        pltpu.make_async_copy(
                            g_hbm.at[src_v.at[j]], gbuf.at[b],
                            sems[b]).wait()
                        pltpu.sync_copy(
                            gbuf.at[b], acc_sh.at[dst_v.at[j]], add=True)
                        nxt = j + NB

                        @pl.when(nxt < CP)
                        def _():
                            pltpu.async_copy(
                                g_hbm.at[src_v.at[nxt]], gbuf.at[b],
                                sems[b])
                    return carry

                lax.fori_loop(0, CP // NB, body, 0)

        @pl.when(c == 0)
        def _():
            run_loops(src0_hbm, dst0_hbm, C0 // CP)

        @pl.when(c == 1)
        def _():
            run_loops(src1_hbm, dst1_hbm, C1 // CP)

        plsc.subcore_barrier()

        @pl.when(c == 0)
        def _():
            pltpu.sync_copy(acc_sh.at[rows], acc0_hbm.at[rows])

        @pl.when(c == 1)
        def _():
            pltpu.sync_copy(acc_sh.at[rows], acc1_hbm.at[rows])

    return k(src0, dst0, src1, dst1, g_pad, zrows)


def _dinv(d0_ref, d1_ref):
    d = d0_ref[:, 0:1] + d1_ref[:, 0:1] + 1.0
    return lax.rsqrt(d)


def _deg_spec():
    return pl.BlockSpec((BM, FDIM), lambda i: (i, 0))


def _tc_first(xp, W1, deg0, deg1):
    def body(x_ref, w_ref, d0_ref, d1_ref, g_ref):
        h = jnp.dot(x_ref[...], w_ref[...],
                    preferred_element_type=jnp.float32)
        g_ref[...] = h * _dinv(d0_ref, d1_ref)

    return pl.pallas_call(
        body,
        grid=(NPAD // BM,),
        in_specs=[
            pl.BlockSpec((BM, FDIM), lambda i: (i, 0)),
            pl.BlockSpec((FDIM, FDIM), lambda i: (0, 0)),
            _deg_spec(),
            _deg_spec(),
        ],
        out_specs=pl.BlockSpec((BM, FDIM), lambda i: (i, 0)),
        out_shape=jax.ShapeDtypeStruct((NPAD, FDIM), jnp.float32),
    )(xp, W1, deg0, deg1)


def _tc_mid(a0, a1, g1, deg0, deg1, b1, W2):
    def body(a0_ref, a1_ref, g_ref, d0_ref, d1_ref, b_ref, w_ref, o_ref):
        dinv = _dinv(d0_ref, d1_ref)
        z = dinv * (a0_ref[...] + a1_ref[...] + g_ref[...]) + b_ref[...]
        z = jnp.maximum(z, 0.0)
        h = jnp.dot(z, w_ref[...], preferred_element_type=jnp.float32)
        o_ref[...] = h * dinv

    return pl.pallas_call(
        body,
        grid=(NPAD // BM,),
        in_specs=[
            pl.BlockSpec((BM, FDIM), lambda i: (i, 0)),
            pl.BlockSpec((BM, FDIM), lambda i: (i, 0)),
            pl.BlockSpec((BM, FDIM), lambda i: (i, 0)),
            _deg_spec(),
            _deg_spec(),
            pl.BlockSpec((1, FDIM), lambda i: (0, 0)),
            pl.BlockSpec((FDIM, FDIM), lambda i: (0, 0)),
        ],
        out_specs=pl.BlockSpec((BM, FDIM), lambda i: (i, 0)),
        out_shape=jax.ShapeDtypeStruct((NPAD, FDIM), jnp.float32),
    )(a0, a1, g1, deg0, deg1, b1.reshape(1, FDIM), W2)


def _tc_last(a0, a1, g2, deg0, deg1, b2, Wfc, bfc):
    def body(a0_ref, a1_ref, g_ref, d0_ref, d1_ref, b_ref, w_ref, bf_ref,
             o_ref):
        dinv = _dinv(d0_ref, d1_ref)
        z = dinv * (a0_ref[...] + a1_ref[...] + g_ref[...]) + b_ref[...]
        z = jnp.maximum(z, 0.0)
        o_ref[...] = jnp.sum(z * w_ref[...], axis=1) + bf_ref[0, 0]

    return pl.pallas_call(
        body,
        grid=(NPAD // BM,),
        in_specs=[
            pl.BlockSpec((BM, FDIM), lambda i: (i, 0)),
            pl.BlockSpec((BM, FDIM), lambda i: (i, 0)),
            pl.BlockSpec((BM, FDIM), lambda i: (i, 0)),
            _deg_spec(),
            _deg_spec(),
            pl.BlockSpec((1, FDIM), lambda i: (0, 0)),
            pl.BlockSpec((1, FDIM), lambda i: (0, 0)),
            pl.BlockSpec((1, 1), lambda i: (0, 0)),
        ],
        out_specs=pl.BlockSpec((BM,), lambda i: (i,)),
        out_shape=jax.ShapeDtypeStruct((NPAD,), jnp.float32),
    )(a0, a1, g2, deg0, deg1, b2.reshape(1, FDIM), Wfc.reshape(1, FDIM),
      bfc.reshape(1, 1))


def kernel(x, edge_index, W1, b1, W2, b2, Wfc, bfc):
    n = x.shape[0]
    e = edge_index.shape[1]
    ct = -(-e // (NS * EB))
    c1 = max(CP, int(round(SC1_FRAC * ct / CP)) * CP)
    c0 = -(-(ct - c1) // CP) * CP
    if (c0 + c1) % (2 * CP):
        c0 += CP
    e0 = NS * c0 * EB
    epad = e0 + NS * c1 * EB

    pad_idx = jnp.full((epad - e,), NPAD - 1, jnp.int32)
    src_all = jnp.concatenate([edge_index[0], pad_idx])
    dst_all = jnp.concatenate([edge_index[1], pad_idx])
    src0 = src_all[:e0].reshape(NS, c0, EB)
    dst0 = dst_all[:e0].reshape(NS, c0, EB)
    src1 = src_all[e0:].reshape(NS, c1, EB)
    dst1 = dst_all[e0:].reshape(NS, c1, EB)
    xp = jnp.pad(x, ((0, NPAD - n), (0, 0)))
    zrows = jnp.zeros((NPAD, FDIM), jnp.float32)

    ones_blk = jnp.ones((EB, FDIM), jnp.float32)
    dst_deg = dst_all.reshape(NS, c0 + c1, EB)
    deg0, deg1 = _sc_degree(dst_deg, ones_blk, zrows)
    g1 = _tc_first(xp, W1, deg0, deg1)
    a0, a1 = _sc_accumulate(src0, dst0, src1, dst1, g1, zrows)
    g2 = _tc_mid(a0, a1, g1, deg0, deg1, b1, W2)
    a0b, a1b = _sc_accumulate(src0, dst0, src1, dst1, g2, zrows)
    outp = _tc_last(a0b, a1b, g2, deg0, deg1, b2, Wfc, bfc)
    return outp[:n]
